# baseline (speedup 1.0000x reference)
import functools

import jax
import jax.numpy as jnp
from jax import lax
from jax.experimental import pallas as pl
from jax.experimental.pallas import tpu as pltpu
from jax.experimental.pallas import tpu_sc as plsc

_B, _C, _Z, _Y, _X = 2, 128, 10, 128, 128
_V = _Z * _Y * _X
_N_PTS = 200000

_NC, _NS, _L = 2, 16, 16
_NW = _NC * _NS
_PTS_W = 6256
_CHUNK = 128
_GRP = _CHUNK // _L
_NCHUNK = (_PTS_W + _CHUNK - 1) // _CHUNK
_SPM = _B * _V
_STRIPE = _SPM // _NS
_ZB = 2048
_HALF = _V // _NS

_ROUND_MAGIC = 8388608.0


def _sc_scatter_body(pts_hbm, out_hbm, spmem, pts_v, idx_v, ones_v, zbuf_v,
                     sem_pts, sem_sc):
    c = lax.axis_index("c")
    s = lax.axis_index("s")
    wid = c * _NS + s

    pt_start = jnp.minimum(wid * _PTS_W, _N_PTS - _PTS_W)
    pts_cp = pltpu.async_copy(
        pts_hbm.at[pl.ds(pt_start * 5, _PTS_W * 5)], pts_v, sem_pts)

    def _zinit(i, carry):
        zbuf_v[pl.ds(i * _L, _L)] = jnp.zeros((_L,), jnp.float32)
        return carry

    lax.fori_loop(0, _ZB // _L, _zinit, 0)

    def _oinit(i, carry):
        ones_v[pl.ds(i * _L, _L)] = jnp.ones((_L,), jnp.float32)
        return carry

    lax.fori_loop(0, _CHUNK // _L, _oinit, 0)

    zcps = [pltpu.async_copy(
        zbuf_v, spmem.at[pl.ds(s * _STRIPE + k * _ZB, _ZB)], sem_sc)
        for k in range(_STRIPE // _ZB)]
    for cp in zcps:
        cp.wait()
    plsc.subcore_barrier()
    pts_cp.wait()

    lane = lax.iota(jnp.int32, _L)

    def _chunk(j, carry):
        for g in range(_GRP):
            slot = jnp.minimum(j * _CHUNK + g * _L + lane, _PTS_W - 1)
            base = slot * 5
            bf = plsc.load_gather(pts_v, [base])
            xf = plsc.load_gather(pts_v, [base + 1])
            yf = plsc.load_gather(pts_v, [base + 2])
            zf = plsc.load_gather(pts_v, [base + 3])
            lx = (xf - (-51.2)) / 0.8
            ly = (yf - (-51.2)) / 0.8
            lz = (zf - (-5.0)) / 0.8
            rx = (lx + _ROUND_MAGIC) - _ROUND_MAGIC
            ry = (ly + _ROUND_MAGIC) - _ROUND_MAGIC
            rz = (lz + _ROUND_MAGIC) - _ROUND_MAGIC
            xi = jnp.clip(rx, 0.0, float(_X - 1)).astype(jnp.int32)
            yi = jnp.clip(ry, 0.0, float(_Y - 1)).astype(jnp.int32)
            zi = jnp.clip(rz, 0.0, float(_Z - 1)).astype(jnp.int32)
            bi = bf.astype(jnp.int32)
            flat = ((bi * _Z + zi) * _Y + yi) * _X + xi
            idx_v[j, pl.ds(g * _L, _L)] = flat
        pltpu.async_copy(ones_v, spmem.at[idx_v.at[j]], sem_sc, add=True)
        return carry

    lax.fori_loop(0, _NCHUNK, _chunk, 0)

    def _drain(j, carry):
        pltpu.make_async_copy(ones_v, spmem.at[idx_v.at[0]], sem_sc).wait()
        return carry

    lax.fori_loop(0, _NCHUNK, _drain, 0)
    plsc.subcore_barrier()

    for b in range(_B):
        pltpu.sync_copy(
            spmem.at[pl.ds(b * _V + s * _HALF, _HALF)],
            out_hbm.at[pl.ds(b * (_NC * _V) + c * _V + s * _HALF, _HALF)],
        )


@functools.cache
def _sc_scatter():
    return pl.kernel(
        _sc_scatter_body,
        out_type=jax.ShapeDtypeStruct((_B * _NC * _V,), jnp.float32),
        mesh=plsc.VectorSubcoreMesh(core_axis_name="c", subcore_axis_name="s"),
        scratch_types=[
            pltpu.VMEM_SHARED((_SPM,), jnp.float32),
            pltpu.VMEM((_PTS_W * 5,), jnp.float32),
            pltpu.VMEM((_NCHUNK, _CHUNK), jnp.int32),
            pltpu.VMEM((_CHUNK,), jnp.float32),
            pltpu.VMEM((_ZB,), jnp.float32),
            pltpu.SemaphoreType.DMA,
            pltpu.SemaphoreType.DMA,
        ],
        compiler_params=pltpu.CompilerParams(needs_layout_passes=False),
    )


_VB = 16384
_NVB = _V // _VB
_YT = _VB // _X
_NYT = _Y // _YT
_FVB = 8192
_FYT = _FVB // _X
_FNYT = _Y // _FYT

_DN_T = (((1,), (1,)), ((), ()))
_DN_M = (((1,), (0,)), ((), ()))


def _moments_body(vf_ref, cnt0_ref, cnt1_ref, S2_ref, sums_ref, mask_ref):
    v = pl.program_id(1)
    f = vf_ref[0, :, 0].reshape(_C, _VB)
    m = jnp.where(cnt0_ref[...] + cnt1_ref[...] > 0.0, 1.0, 0.0)
    mask_ref[0, 0] = m
    H = jnp.concatenate([f, f * m[None, :]], axis=0)
    S2_blk = lax.dot_general(H, H, _DN_T, preferred_element_type=jnp.float32)

    @pl.when(v == 0)
    def _():
        S2_ref[...] = jnp.zeros_like(S2_ref)
        sums_ref[...] = jnp.zeros_like(sums_ref)

    S2_ref[0] += S2_blk
    sums_ref[0, 0] += jnp.sum(f, axis=1)
    sums_ref[0, 1] += jnp.sum(f * m[None, :], axis=1)


def _moments(vf5, cnt):
    nb = 2 * _NVB
    return pl.pallas_call(
        _moments_body,
        grid=(_B, _NVB),
        in_specs=[
            pl.BlockSpec((1, _C, 1, _YT, _X),
                         lambda b, v: (b, 0, v // _NYT, v % _NYT, 0)),
            pl.BlockSpec((_VB,), lambda b, v: (b * nb + v,)),
            pl.BlockSpec((_VB,), lambda b, v: (b * nb + _NVB + v,)),
        ],
        out_specs=[
            pl.BlockSpec((1, 2 * _C, 2 * _C), lambda b, v: (b, 0, 0)),
            pl.BlockSpec((1, 2, _C), lambda b, v: (b, 0, 0)),
            pl.BlockSpec((1, 1, _VB), lambda b, v: (b, 0, v)),
        ],
        out_shape=[
            jax.ShapeDtypeStruct((_B, 2 * _C, 2 * _C), jnp.float32),
            jax.ShapeDtypeStruct((_B, 2, _C), jnp.float32),
            jax.ShapeDtypeStruct((_B, 1, _V), jnp.float32),
        ],
        compiler_params=pltpu.CompilerParams(
            dimension_semantics=("arbitrary", "arbitrary")),
    )(vf5, cnt, cnt)


def _prep_body(S2_ref, sums_ref, sew_ref, seb_ref, sqw_ref, sqb_ref,
               gam_ref, bet_ref, A12_ref, bias_ref):
    sqb = sqb_ref[0]
    gam = gam_ref[0]
    bet = bet_ref[0]
    seb = seb_ref[0]
    sqw = sqw_ref[...]
    sew = sew_ref[...]
    inv_v = 1.0 / _V
    inv_bv = 1.0 / (_B * _V)

    a12s = []
    mean_acc = jnp.zeros((_C,), jnp.float32)
    e2_acc = jnp.zeros((_C,), jnp.float32)
    for b in range(_B):
        s1 = sums_ref[b, 0]
        sm1 = sums_ref[b, 1]
        att = jax.nn.sigmoid(
            jnp.sum(sew[:, :_C] * (sm1 * inv_v)[None, :], axis=1)
            + jnp.sum(sew[:, _C:] * (s1 * inv_v)[None, :], axis=1) + seb)
        a1 = sqw[:, :_C] * att[None, :_C]
        a2 = sqw[:, _C:] * att[None, _C:]
        a12s.append((a1, a2))
        Sb = S2_ref[b, :_C, :_C]
        Smb = S2_ref[b, _C:, _C:]
        t_lin = (jnp.sum(a2 * s1[None, :], axis=1)
                 + jnp.sum(a1 * sm1[None, :], axis=1))
        mean_acc += t_lin
        a2S = lax.dot_general(a2, Sb, _DN_M, preferred_element_type=jnp.float32)
        a1Sm = lax.dot_general(a1, Smb, _DN_M, preferred_element_type=jnp.float32)
        a2Sm = lax.dot_general(a2, Smb, _DN_M, preferred_element_type=jnp.float32)
        e2_acc += (jnp.sum(a2S * a2, axis=1)
                   + jnp.sum(a1Sm * a1, axis=1)
                   + 2.0 * jnp.sum(a2Sm * a1, axis=1)
                   + 2.0 * sqb * t_lin)

    mean = mean_acc * inv_bv + sqb
    e2 = e2_acc * inv_bv + sqb * sqb
    var = e2 - mean * mean
    alpha = gam * lax.rsqrt(var + 1e-5)
    for b in range(_B):
        a1, a2 = a12s[b]
        A12_ref[b] = jnp.concatenate(
            [alpha[:, None] * a2, alpha[:, None] * a1], axis=0)
    bias_ref[...] = (alpha * (sqb - mean) + bet)[:, None]


def _prep(S2, sums, se_w, se_b, sq_w, sq_b, gamma, beta):
    return pl.pallas_call(
        _prep_body,
        out_shape=[
            jax.ShapeDtypeStruct((_B, 2 * _C, _C), jnp.float32),
            jax.ShapeDtypeStruct((_C, 1), jnp.float32),
        ],
    )(S2, sums, se_w, se_b.reshape(1, -1), sq_w, sq_b.reshape(1, -1),
      gamma.reshape(1, -1), beta.reshape(1, -1))


def _fused_out_body(vf_ref, mask_ref, A12_ref, bias_ref, out_ref):
    f = vf_ref[0, :, 0].reshape(_C, _FVB)
    m = mask_ref[0, 0]
    uw = lax.dot_general(A12_ref[0], f, _DN_M,
                         preferred_element_type=jnp.float32)
    conv = uw[:_C] + uw[_C:] * m[None, :] + bias_ref[...]
    o = jnp.maximum(conv, 0.0)
    out_ref[0, :, 0] = o.reshape(_C, _FYT, _X)


def _fused_out(vf5, mask, A12p, bias_col):
    return pl.pallas_call(
        _fused_out_body,
        grid=(_B, _Z, _FNYT),
        in_specs=[
            pl.BlockSpec((1, _C, 1, _FYT, _X), lambda b, z, y: (b, 0, z, y, 0)),
            pl.BlockSpec((1, 1, _FVB), lambda b, z, y: (b, 0, z * _FNYT + y)),
            pl.BlockSpec((1, 2 * _C, _C), lambda b, z, y: (b, 0, 0)),
            pl.BlockSpec((_C, 1), lambda b, z, y: (0, 0)),
        ],
        out_specs=pl.BlockSpec((1, _C, 1, _FYT, _X),
                               lambda b, z, y: (b, 0, z, y, 0)),
        out_shape=jax.ShapeDtypeStruct((_B, _C, _Z, _Y, _X), jnp.float32),
        compiler_params=pltpu.CompilerParams(
            dimension_semantics=("arbitrary", "arbitrary", "arbitrary")),
    )(vf5, mask, A12p, bias_col)


def kernel(voxel_features, points, se_w, se_b, sq_w, sq_b, gamma, beta):
    cnt = _sc_scatter()(points.reshape(-1))
    S2, sums, mask = _moments(voxel_features, cnt)
    A12p, bias_col = _prep(S2, sums, se_w, se_b, sq_w, sq_b, gamma, beta)
    return _fused_out(voxel_features, mask, A12p, bias_col)


# speedup vs baseline: 1.0230x; 1.0230x over previous
import functools

import jax
import jax.numpy as jnp
from jax import lax
from jax.experimental import pallas as pl
from jax.experimental.pallas import tpu as pltpu
from jax.experimental.pallas import tpu_sc as plsc

_B, _C, _Z, _Y, _X = 2, 128, 10, 128, 128
_V = _Z * _Y * _X
_N_PTS = 200000

_NC, _NS, _L = 2, 16, 16
_NW = _NC * _NS
_PTS_W = 6256
_CHUNK = 128
_GRP = _CHUNK // _L
_NCHUNK = (_PTS_W + _CHUNK - 1) // _CHUNK
_SPM = _B * _V
_STRIPE = _SPM // _NS
_ZB = 2048
_HALF = _V // _NS

_ROUND_MAGIC = 8388608.0


def _sc_scatter_body(pts_hbm, out_hbm, spmem, pts_v, idx_v, ones_v, zbuf_v,
                     sem_pts, sem_sc):
    c = lax.axis_index("c")
    s = lax.axis_index("s")
    wid = c * _NS + s

    pt_start = jnp.minimum(wid * _PTS_W, _N_PTS - _PTS_W)
    pts_cp = pltpu.async_copy(
        pts_hbm.at[pl.ds(pt_start * 5, _PTS_W * 5)], pts_v, sem_pts)

    def _zinit(i, carry):
        zbuf_v[pl.ds(i * _L, _L)] = jnp.zeros((_L,), jnp.float32)
        return carry

    lax.fori_loop(0, _ZB // _L, _zinit, 0)

    def _oinit(i, carry):
        ones_v[pl.ds(i * _L, _L)] = jnp.ones((_L,), jnp.float32)
        return carry

    lax.fori_loop(0, _CHUNK // _L, _oinit, 0)

    zcps = [pltpu.async_copy(
        zbuf_v, spmem.at[pl.ds(s * _STRIPE + k * _ZB, _ZB)], sem_sc)
        for k in range(_STRIPE // _ZB)]
    for cp in zcps:
        cp.wait()
    plsc.subcore_barrier()
    pts_cp.wait()

    lane = lax.iota(jnp.int32, _L)

    def _chunk(j, carry):
        for g in range(_GRP):
            slot = jnp.minimum(j * _CHUNK + g * _L + lane, _PTS_W - 1)
            base = slot * 5
            bf = plsc.load_gather(pts_v, [base])
            xf = plsc.load_gather(pts_v, [base + 1])
            yf = plsc.load_gather(pts_v, [base + 2])
            zf = plsc.load_gather(pts_v, [base + 3])
            lx = (xf - (-51.2)) / 0.8
            ly = (yf - (-51.2)) / 0.8
            lz = (zf - (-5.0)) / 0.8
            rx = (lx + _ROUND_MAGIC) - _ROUND_MAGIC
            ry = (ly + _ROUND_MAGIC) - _ROUND_MAGIC
            rz = (lz + _ROUND_MAGIC) - _ROUND_MAGIC
            xi = jnp.clip(rx, 0.0, float(_X - 1)).astype(jnp.int32)
            yi = jnp.clip(ry, 0.0, float(_Y - 1)).astype(jnp.int32)
            zi = jnp.clip(rz, 0.0, float(_Z - 1)).astype(jnp.int32)
            bi = bf.astype(jnp.int32)
            flat = ((bi * _Z + zi) * _Y + yi) * _X + xi
            idx_v[j, pl.ds(g * _L, _L)] = flat
        pltpu.async_copy(ones_v, spmem.at[idx_v.at[j]], sem_sc, add=True)
        return carry

    lax.fori_loop(0, _NCHUNK, _chunk, 0)

    def _drain(j, carry):
        pltpu.make_async_copy(ones_v, spmem.at[idx_v.at[0]], sem_sc).wait()
        return carry

    lax.fori_loop(0, _NCHUNK, _drain, 0)
    plsc.subcore_barrier()

    for b in range(_B):
        pltpu.sync_copy(
            spmem.at[pl.ds(b * _V + s * _HALF, _HALF)],
            out_hbm.at[pl.ds(b * (_NC * _V) + c * _V + s * _HALF, _HALF)],
        )


@functools.cache
def _sc_scatter():
    return pl.kernel(
        _sc_scatter_body,
        out_type=jax.ShapeDtypeStruct((_B * _NC * _V,), jnp.float32),
        mesh=plsc.VectorSubcoreMesh(core_axis_name="c", subcore_axis_name="s"),
        scratch_types=[
            pltpu.VMEM_SHARED((_SPM,), jnp.float32),
            pltpu.VMEM((_PTS_W * 5,), jnp.float32),
            pltpu.VMEM((_NCHUNK, _CHUNK), jnp.int32),
            pltpu.VMEM((_CHUNK,), jnp.float32),
            pltpu.VMEM((_ZB,), jnp.float32),
            pltpu.SemaphoreType.DMA,
            pltpu.SemaphoreType.DMA,
        ],
        compiler_params=pltpu.CompilerParams(needs_layout_passes=False),
    )


_VB = 16384
_NVB = _V // _VB
_YT = _VB // _X
_NYT = _Y // _YT
_FVB = 16384
_FYT = _FVB // _X
_FNYT = _Y // _FYT

_DN_T = (((1,), (1,)), ((), ()))
_DN_M = (((1,), (0,)), ((), ()))


def _moments_body(vf_ref, cnt0_ref, cnt1_ref, S2_ref, sums_ref, mask_ref):
    v = pl.program_id(1)
    f = vf_ref[0, :, 0].reshape(_C, _VB)
    m = jnp.where(cnt0_ref[...] + cnt1_ref[...] > 0.0, 1.0, 0.0)
    mask_ref[0, 0] = m
    H = jnp.concatenate([f, f * m[None, :]], axis=0)
    S2_blk = lax.dot_general(H, H, _DN_T, preferred_element_type=jnp.float32)

    @pl.when(v == 0)
    def _():
        S2_ref[...] = jnp.zeros_like(S2_ref)
        sums_ref[...] = jnp.zeros_like(sums_ref)

    S2_ref[0] += S2_blk
    sums_ref[0, 0] += jnp.sum(f, axis=1)
    sums_ref[0, 1] += jnp.sum(f * m[None, :], axis=1)


def _moments(vf5, cnt):
    nb = 2 * _NVB
    return pl.pallas_call(
        _moments_body,
        grid=(_B, _NVB),
        in_specs=[
            pl.BlockSpec((1, _C, 1, _YT, _X),
                         lambda b, v: (b, 0, v // _NYT, v % _NYT, 0)),
            pl.BlockSpec((_VB,), lambda b, v: (b * nb + v,)),
            pl.BlockSpec((_VB,), lambda b, v: (b * nb + _NVB + v,)),
        ],
        out_specs=[
            pl.BlockSpec((1, 2 * _C, 2 * _C), lambda b, v: (b, 0, 0)),
            pl.BlockSpec((1, 2, _C), lambda b, v: (b, 0, 0)),
            pl.BlockSpec((1, 1, _VB), lambda b, v: (b, 0, v)),
        ],
        out_shape=[
            jax.ShapeDtypeStruct((_B, 2 * _C, 2 * _C), jnp.float32),
            jax.ShapeDtypeStruct((_B, 2, _C), jnp.float32),
            jax.ShapeDtypeStruct((_B, 1, _V), jnp.float32),
        ],
        compiler_params=pltpu.CompilerParams(
            dimension_semantics=("arbitrary", "arbitrary")),
    )(vf5, cnt, cnt)


def _prep_body(S2_ref, sums_ref, sew_ref, seb_ref, sqw_ref, sqb_ref,
               gam_ref, bet_ref, A12_ref, bias_ref):
    sqb = sqb_ref[0]
    gam = gam_ref[0]
    bet = bet_ref[0]
    seb = seb_ref[0]
    sqw = sqw_ref[...]
    sew = sew_ref[...]
    inv_v = 1.0 / _V
    inv_bv = 1.0 / (_B * _V)

    a12s = []
    mean_acc = jnp.zeros((_C,), jnp.float32)
    e2_acc = jnp.zeros((_C,), jnp.float32)
    for b in range(_B):
        s1 = sums_ref[b, 0]
        sm1 = sums_ref[b, 1]
        att = jax.nn.sigmoid(
            jnp.sum(sew[:, :_C] * (sm1 * inv_v)[None, :], axis=1)
            + jnp.sum(sew[:, _C:] * (s1 * inv_v)[None, :], axis=1) + seb)
        a1 = sqw[:, :_C] * att[None, :_C]
        a2 = sqw[:, _C:] * att[None, _C:]
        a12s.append((a1, a2))
        Sb = S2_ref[b, :_C, :_C]
        Smb = S2_ref[b, _C:, _C:]
        t_lin = (jnp.sum(a2 * s1[None, :], axis=1)
                 + jnp.sum(a1 * sm1[None, :], axis=1))
        mean_acc += t_lin
        a2S = lax.dot_general(a2, Sb, _DN_M, preferred_element_type=jnp.float32)
        a1Sm = lax.dot_general(a1, Smb, _DN_M, preferred_element_type=jnp.float32)
        a2Sm = lax.dot_general(a2, Smb, _DN_M, preferred_element_type=jnp.float32)
        e2_acc += (jnp.sum(a2S * a2, axis=1)
                   + jnp.sum(a1Sm * a1, axis=1)
                   + 2.0 * jnp.sum(a2Sm * a1, axis=1)
                   + 2.0 * sqb * t_lin)

    mean = mean_acc * inv_bv + sqb
    e2 = e2_acc * inv_bv + sqb * sqb
    var = e2 - mean * mean
    alpha = gam * lax.rsqrt(var + 1e-5)
    for b in range(_B):
        a1, a2 = a12s[b]
        A12_ref[b] = jnp.concatenate(
            [alpha[:, None] * a2, alpha[:, None] * a1], axis=0)
    bias_ref[...] = (alpha * (sqb - mean) + bet)[:, None]


def _prep(S2, sums, se_w, se_b, sq_w, sq_b, gamma, beta):
    return pl.pallas_call(
        _prep_body,
        out_shape=[
            jax.ShapeDtypeStruct((_B, 2 * _C, _C), jnp.float32),
            jax.ShapeDtypeStruct((_C, 1), jnp.float32),
        ],
    )(S2, sums, se_w, se_b.reshape(1, -1), sq_w, sq_b.reshape(1, -1),
      gamma.reshape(1, -1), beta.reshape(1, -1))


def _fused_out_body(vf_ref, mask_ref, A12_ref, bias_ref, out_ref):
    f = vf_ref[0, :, 0].reshape(_C, _FVB)
    m = mask_ref[0, 0]
    uw = lax.dot_general(A12_ref[0], f, _DN_M,
                         preferred_element_type=jnp.float32)
    conv = uw[:_C] + uw[_C:] * m[None, :] + bias_ref[...]
    o = jnp.maximum(conv, 0.0)
    out_ref[0, :, 0] = o.reshape(_C, _FYT, _X)


def _fused_out(vf5, mask, A12p, bias_col):
    return pl.pallas_call(
        _fused_out_body,
        grid=(_B, _Z, _FNYT),
        in_specs=[
            pl.BlockSpec((1, _C, 1, _FYT, _X), lambda b, z, y: (b, 0, z, y, 0)),
            pl.BlockSpec((1, 1, _FVB), lambda b, z, y: (b, 0, z * _FNYT + y)),
            pl.BlockSpec((1, 2 * _C, _C), lambda b, z, y: (b, 0, 0)),
            pl.BlockSpec((_C, 1), lambda b, z, y: (0, 0)),
        ],
        out_specs=pl.BlockSpec((1, _C, 1, _FYT, _X),
                               lambda b, z, y: (b, 0, z, y, 0)),
        out_shape=jax.ShapeDtypeStruct((_B, _C, _Z, _Y, _X), jnp.float32),
        compiler_params=pltpu.CompilerParams(
            dimension_semantics=("arbitrary", "arbitrary", "arbitrary")),
    )(vf5, mask, A12p, bias_col)


def kernel(voxel_features, points, se_w, se_b, sq_w, sq_b, gamma, beta):
    cnt = _sc_scatter()(points.reshape(-1))
    S2, sums, mask = _moments(voxel_features, cnt)
    A12p, bias_col = _prep(S2, sums, se_w, se_b, sq_w, sq_b, gamma, beta)
    return _fused_out(voxel_features, mask, A12p, bias_col)
